# baseline (speedup 1.0000x reference)
import jax
import jax.numpy as jnp
from jax.experimental import pallas as pl
from jax.experimental.pallas import tpu as pltpu

BAG = 64
CLS = 128
N = 16384
BM = 2048
GRID = N // BM


def _tc_body(yt_ref, yp_ref, th_ref, out_ref, acc_ref, cnt_ref):
    i = pl.program_id(0)
    ids = yt_ref[0]
    oh = (jax.lax.broadcasted_iota(jnp.int32, (BAG, 1), 0) == ids).astype(
        jnp.float32
    )
    psum = jax.lax.dot_general(
        oh, yp_ref[...], (((1,), (0,)), ((), ())),
        preferred_element_type=jnp.float32,
    )
    pcnt = jnp.sum(oh, axis=1, keepdims=True)

    @pl.when(i == 0)
    def _():
        acc_ref[...] = psum
        cnt_ref[...] = pcnt

    @pl.when(i > 0)
    def _():
        acc_ref[...] += psum
        cnt_ref[...] += pcnt

    @pl.when(i == GRID - 1)
    def _():
        means = acc_ref[...] / cnt_ref[...]
        theta_c = jnp.clip(th_ref[...], 1e-07, 1.0 - 1e-07)
        m = jnp.max(means, axis=-1, keepdims=True)
        e = jnp.exp(means - m)
        s = jnp.sum(e, axis=-1, keepdims=True)
        sm = e / s
        loss = -theta_c * jnp.log(sm + 1e-07)
        out_ref[0, 0] = jnp.sum(loss)


def kernel(y_true, y_pred, theta):
    yt = y_true.astype(jnp.int32).reshape(GRID, 1, BM)
    out = pl.pallas_call(
        _tc_body,
        grid=(GRID,),
        out_shape=jax.ShapeDtypeStruct((1, 1), jnp.float32),
        in_specs=[
            pl.BlockSpec((1, 1, BM), lambda i: (i, 0, 0)),
            pl.BlockSpec((BM, CLS), lambda i: (i, 0)),
            pl.BlockSpec((BAG, 1), lambda i: (0, 0)),
        ],
        out_specs=pl.BlockSpec((1, 1), lambda i: (0, 0),
                               memory_space=pltpu.SMEM),
        scratch_shapes=[
            pltpu.VMEM((BAG, CLS), jnp.float32),
            pltpu.VMEM((BAG, 1), jnp.float32),
        ],
    )(yt, y_pred, theta.reshape(BAG, 1))
    return out[0, 0]
